# baseline (speedup 1.0000x reference)
import functools

import jax
import jax.numpy as jnp
from jax import lax
from jax.experimental import pallas as pl
from jax.experimental.pallas import tpu as pltpu
from jax.experimental.pallas import tpu_sc as plsc

N = 50000
E = 1600000
F_IN = 1433
H = 128
C = 7

NT = 16
NC = 2
LW = 128
KC = 8
KC1 = 4
R = 12544
E_PAD = R * LW
R_TILE = R // NT
R_HALF_TILE = R // NC // NT
ACC_ROWS = 51200
ZROWS = ACC_ROWS // NT
N_PAD = 50048
OUT_TILE = N_PAD // NT

BM = 1000


def _mesh():
    return plsc.VectorSubcoreMesh(core_axis_name="c", subcore_axis_name="s")


def _deg_body(dstr_hbm, zeros_hbm, out_hbm, ones_v, dst_v, acc, sem):
    cid = lax.axis_index("c")
    tid = lax.axis_index("s")
    o16 = jnp.ones((16,), jnp.float32)
    for q in range(LW // 16):
        ones_v[pl.ds(q * 16, 16)] = o16

    pltpu.sync_copy(zeros_hbm, acc.at[pl.ds(tid * ZROWS, ZROWS)])
    plsc.subcore_barrier()

    def chunk(ci, _):
        row0 = cid * (R // NC) + tid * R_HALF_TILE + ci * KC
        pltpu.sync_copy(dstr_hbm.at[pl.ds(row0, KC)], dst_v)
        cps = [
            pltpu.async_copy(ones_v, acc.at[dst_v.at[j]], sem, add=True)
            for j in range(KC)
        ]
        for cp in cps:
            cp.wait()
        return 0
    lax.fori_loop(0, R_HALF_TILE // KC, chunk, 0, unroll=False)
    plsc.subcore_barrier()

    pltpu.sync_copy(
        acc.at[pl.ds(tid * ZROWS, ZROWS)],
        out_hbm.at[pl.ds(cid * ACC_ROWS + tid * ZROWS, ZROWS)],
    )


def _deg_call(dstr, zeros1):
    return pl.kernel(
        _deg_body,
        out_type=jax.ShapeDtypeStruct((NC * ACC_ROWS,), jnp.float32),
        mesh=_mesh(),
        scratch_types=[
            pltpu.VMEM((LW,), jnp.float32),
            pltpu.VMEM((KC, LW), jnp.int32),
            pltpu.VMEM_SHARED((ACC_ROWS,), jnp.float32),
            pltpu.SemaphoreType.DMA,
        ],
        compiler_params=pltpu.CompilerParams(use_tc_tiling_on_sc=False),
        name="sc_deg_hist",
    )(dstr, zeros1)


def _agg1_body(h1s_hbm, edges2_hbm, zeros_hbm, out_hbm,
               idx_v, rows_v, acc, sem_i, sem_g, sem_sa, sem_sb):
    cid = lax.axis_index("c")
    tid = lax.axis_index("s")
    NIT = R_TILE // 4
    sem_s = (sem_sa, sem_sb)

    def stage(X, s_idx, w):
        base = 2 * (s_idx * R + tid * R_TILE + 2 * w)
        pltpu.async_copy(edges2_hbm.at[pl.ds(base, 4)], idx_v.at[X], sem_i)

    def stage_wait(X):
        pltpu.make_async_copy(
            edges2_hbm.at[pl.ds(0, 4)], idx_v.at[X], sem_i).wait()

    def gathers(X):
        for k in range(2):
            pltpu.async_copy(
                h1s_hbm.at[idx_v.at[X, 2 * k]], rows_v.at[X, k], sem_g)

    def gathers_wait(X):
        for k in range(2):
            pltpu.make_async_copy(
                h1s_hbm.at[pl.ds(0, LW)], rows_v.at[X, k], sem_g).wait()

    def scatters(X):
        for k in range(2):
            pltpu.async_copy(
                rows_v.at[X, k], acc.at[idx_v.at[X, 2 * k + 1]], sem_s[X],
                add=True)

    def scatters_wait(X):
        for k in range(2):
            pltpu.make_async_copy(
                h1s_hbm.at[pl.ds(0, LW)], rows_v.at[X, k], sem_s[X]).wait()

    for p in range(2):
        s_idx = cid * 2 + p

        pltpu.sync_copy(zeros_hbm, acc.at[pl.ds(tid * ZROWS, ZROWS)])
        plsc.subcore_barrier()

        stage(0, s_idx, 0)
        stage_wait(0)
        gathers(0)
        stage(1, s_idx, 1)

        def pipe(i, _):
            gathers_wait(0)
            scatters(0)

            @pl.when(i > 0)
            def _():
                scatters_wait(1)
                stage(1, s_idx, 2 * i + 1)
            stage_wait(1)
            gathers(1)
            gathers_wait(1)
            scatters(1)
            scatters_wait(0)

            @pl.when(i < NIT - 1)
            def _():
                stage(0, s_idx, 2 * i + 2)
                stage_wait(0)
                gathers(0)
            return 0
        lax.fori_loop(0, NIT, pipe, 0, unroll=False)
        scatters_wait(1)
        plsc.subcore_barrier()

        pltpu.sync_copy(
            acc.at[pl.ds(tid * OUT_TILE, OUT_TILE)],
            out_hbm.at[pl.ds(s_idx * N_PAD + tid * OUT_TILE, OUT_TILE)],
        )
        plsc.subcore_barrier()


def _agg1_call(h1s2d, edges2, zeros32):
    return pl.kernel(
        _agg1_body,
        out_type=jax.ShapeDtypeStruct((4 * N_PAD, 32), jnp.float32),
        mesh=_mesh(),
        scratch_types=[
            pltpu.VMEM((2, 4, LW), jnp.int32),
            pltpu.VMEM((2, 2, LW, 32), jnp.float32),
            pltpu.VMEM_SHARED((ACC_ROWS, 32), jnp.float32),
            pltpu.SemaphoreType.DMA,
            pltpu.SemaphoreType.DMA,
            pltpu.SemaphoreType.DMA,
            pltpu.SemaphoreType.DMA,
        ],
        compiler_params=pltpu.CompilerParams(use_tc_tiling_on_sc=False),
        name="sc_agg1",
    )(h1s2d, edges2, zeros32)


def _agg2_body(h2p_hbm, srcr_hbm, dstr_hbm, zeros_hbm, out_hbm,
               src_v, dst_v, rows_v, acc, sem, sem2):
    cid = lax.axis_index("c")
    tid = lax.axis_index("s")

    pltpu.sync_copy(zeros_hbm, acc.at[pl.ds(tid * ZROWS, ZROWS)])
    plsc.subcore_barrier()

    def chunk(ci, _):
        row0 = cid * (R // NC) + tid * R_HALF_TILE + ci * KC
        pltpu.sync_copy(srcr_hbm.at[pl.ds(row0, KC)], src_v)
        pltpu.sync_copy(dstr_hbm.at[pl.ds(row0, KC)], dst_v)
        cps = [
            pltpu.async_copy(h2p_hbm.at[src_v.at[j]], rows_v.at[j], sem)
            for j in range(KC)
        ]
        for cp in cps:
            cp.wait()
        sps = [
            pltpu.async_copy(rows_v.at[j], acc.at[dst_v.at[j]], sem2, add=True)
            for j in range(KC)
        ]
        for sp in sps:
            sp.wait()
        return 0
    lax.fori_loop(0, R_HALF_TILE // KC, chunk, 0, unroll=False)
    plsc.subcore_barrier()

    pltpu.sync_copy(
        acc.at[pl.ds(tid * OUT_TILE, OUT_TILE)],
        out_hbm.at[pl.ds(cid * N_PAD + tid * OUT_TILE, OUT_TILE)],
    )


def _agg2_call(h2p, srcr, dstr, zeros8):
    return pl.kernel(
        _agg2_body,
        out_type=jax.ShapeDtypeStruct((NC * N_PAD, 8), jnp.float32),
        mesh=_mesh(),
        scratch_types=[
            pltpu.VMEM((KC, LW), jnp.int32),
            pltpu.VMEM((KC, LW), jnp.int32),
            pltpu.VMEM((KC, LW, 8), jnp.float32),
            pltpu.VMEM_SHARED((ACC_ROWS, 8), jnp.float32),
            pltpu.SemaphoreType.DMA,
            pltpu.SemaphoreType.DMA,
        ],
        compiler_params=pltpu.CompilerParams(use_tc_tiling_on_sc=False),
        name="sc_agg2",
    )(h2p, srcr, dstr, zeros8)


def _mm1_body(x_ref, w_ref, o_ref):
    o_ref[...] = jnp.dot(x_ref[...], w_ref[...],
                         preferred_element_type=jnp.float32)


def _mm1_call(x, W1):
    return pl.pallas_call(
        _mm1_body,
        grid=(N // BM,),
        in_specs=[
            pl.BlockSpec((BM, F_IN), lambda i: (i, 0)),
            pl.BlockSpec((F_IN, H), lambda i: (0, 0)),
        ],
        out_specs=pl.BlockSpec((BM, H), lambda i: (i, 0)),
        out_shape=jax.ShapeDtypeStruct((N, H), jnp.float32),
        name="tc_mm1",
    )(x, W1)


def _scale_body(h1_ref, deg_ref, h1s_ref, dinv_ref):
    deg = deg_ref[:, 0:1] + deg_ref[:, 1:2] + 1.0
    dv = lax.rsqrt(deg)
    dinv_ref[...] = dv
    hp = h1_ref[...] * dv
    for s in range(4):
        h1s_ref[s] = hp[:, 32 * s:32 * s + 32]


def _scale_call(h1, deg2t):
    return pl.pallas_call(
        _scale_body,
        grid=(N // BM,),
        in_specs=[
            pl.BlockSpec((BM, H), lambda i: (i, 0)),
            pl.BlockSpec((BM, 2), lambda i: (i, 0)),
        ],
        out_specs=[
            pl.BlockSpec((4, BM, 32), lambda i: (0, i, 0)),
            pl.BlockSpec((BM, 1), lambda i: (i, 0)),
        ],
        out_shape=[
            jax.ShapeDtypeStruct((4, N, 32), jnp.float32),
            jax.ShapeDtypeStruct((N, 1), jnp.float32),
        ],
        name="tc_scale_slice",
    )(h1, deg2t)


def _layer2_body(agg_ref, h1s_ref, dinv_ref, b1_ref, w2_ref, o_ref):
    dv = dinv_ref[...]
    a = jnp.concatenate(
        [agg_ref[s] + h1s_ref[s] for s in range(4)], axis=1)
    z1 = jnp.maximum(a * dv + b1_ref[...][None, :], 0.0)
    h2 = jnp.dot(z1, w2_ref[...], preferred_element_type=jnp.float32)
    o_ref[...] = h2 * dv


def _layer2_call(agg1, h1s, dinv, b1, W2p):
    return pl.pallas_call(
        _layer2_body,
        grid=(N // BM,),
        in_specs=[
            pl.BlockSpec((4, BM, 32), lambda i: (0, i, 0)),
            pl.BlockSpec((4, BM, 32), lambda i: (0, i, 0)),
            pl.BlockSpec((BM, 1), lambda i: (i, 0)),
            pl.BlockSpec((H,), lambda i: (0,)),
            pl.BlockSpec((H, 8), lambda i: (0, 0)),
        ],
        out_specs=pl.BlockSpec((BM, 8), lambda i: (i, 0)),
        out_shape=jax.ShapeDtypeStruct((N, 8), jnp.float32),
        name="tc_layer2",
    )(agg1, h1s, dinv, b1, W2p)


def _final_body(agg2_ref, h2p_ref, dinv_ref, b2_ref, o_ref):
    s = agg2_ref[0] + agg2_ref[1] + h2p_ref[...]
    o_ref[...] = s * dinv_ref[...] + b2_ref[...][None, :]


def _final_call(agg2, h2p, dinv, b2p):
    return pl.pallas_call(
        _final_body,
        grid=(N // BM,),
        in_specs=[
            pl.BlockSpec((2, BM, 8), lambda i: (0, i, 0)),
            pl.BlockSpec((BM, 8), lambda i: (i, 0)),
            pl.BlockSpec((BM, 1), lambda i: (i, 0)),
            pl.BlockSpec((8,), lambda i: (0,)),
        ],
        out_specs=pl.BlockSpec((BM, 8), lambda i: (i, 0)),
        out_shape=jax.ShapeDtypeStruct((N, 8), jnp.float32),
        name="tc_final",
    )(agg2, h2p, dinv, b2p)


def kernel(x, edge_index, y, W1, b1, W2, b2):
    ei = edge_index.astype(jnp.int32)
    pad = E_PAD - E
    srcr = jnp.concatenate(
        [ei[0], jnp.zeros((pad,), jnp.int32)]).reshape(R, LW)
    dstr = jnp.concatenate(
        [ei[1], jnp.full((pad,), N, jnp.int32)]).reshape(R, LW)
    W2p = jnp.pad(W2, ((0, 0), (0, 1)))
    b2p = jnp.pad(b2, (0, 1))
    zeros1 = jnp.zeros((ZROWS,), jnp.float32)
    zeros32 = jnp.zeros((ZROWS, 32), jnp.float32)
    zeros8 = jnp.zeros((ZROWS, 8), jnp.float32)

    h1 = _mm1_call(x, W1)
    degflat = _deg_call(dstr, zeros1)
    deg2t = degflat.reshape(NC, ACC_ROWS)[:, :N].T
    h1s, dinv = _scale_call(h1, deg2t)
    srcr4 = (srcr[None] + (jnp.arange(4, dtype=jnp.int32) * N)[:, None, None])
    edges2 = jnp.stack(
        [srcr4, jnp.broadcast_to(dstr, (4, R, LW))], axis=2
    ).reshape(8 * R, LW)
    agg1 = _agg1_call(h1s.reshape(4 * N, 32), edges2,
                      zeros32).reshape(4, N_PAD, 32)[:, :N, :]
    h2p = _layer2_call(agg1, h1s, dinv, b1, W2p)
    agg2 = _agg2_call(h2p, srcr, dstr, zeros8).reshape(NC, N_PAD, 8)[:, :N, :]
    out8 = _final_call(agg2, h2p, dinv, b2p)
    return out8[:, :C]


# speedup vs baseline: 1.0641x; 1.0641x over previous
import jax
import jax.numpy as jnp
from jax import lax
from jax.experimental import pallas as pl
from jax.experimental.pallas import tpu as pltpu
from jax.experimental.pallas import tpu_sc as plsc

N = 50000
E = 1600000
F_IN = 1433
H = 128
C = 7

NT = 16
NC = 2
LW = 128
KC = 8
R = 12544
E_PAD = R * LW
R_TILE = R // NT
R_HALF_TILE = R // NC // NT
ACC_ROWS = 51200
ZROWS = ACC_ROWS // NT
OUT_TILE = N // NT

BM = 1000
NB = N // BM


def _mesh():
    return plsc.VectorSubcoreMesh(core_axis_name="c", subcore_axis_name="s")


_SC_PARAMS = pltpu.CompilerParams(use_tc_tiling_on_sc=False)


def _deg_body(dstr_hbm, zeros_hbm, out_hbm, ones_v, dst_v, acc, sem):
    cid = lax.axis_index("c")
    tid = lax.axis_index("s")
    o16 = jnp.ones((16,), jnp.float32)
    for q in range(LW // 16):
        ones_v[pl.ds(q * 16, 16)] = o16

    pltpu.sync_copy(zeros_hbm, acc.at[pl.ds(tid * ZROWS, ZROWS)])
    plsc.subcore_barrier()

    def chunk(ci, _):
        row0 = cid * (R // NC) + tid * R_HALF_TILE + ci * KC
        pltpu.sync_copy(dstr_hbm.at[pl.ds(row0, KC)], dst_v)
        cps = [
            pltpu.async_copy(ones_v, acc.at[dst_v.at[j]], sem, add=True)
            for j in range(KC)
        ]
        for cp in cps:
            cp.wait()
        return 0
    lax.fori_loop(0, R_HALF_TILE // KC, chunk, 0, unroll=False)
    plsc.subcore_barrier()

    pltpu.sync_copy(
        acc.at[pl.ds(tid * ZROWS, ZROWS)],
        out_hbm.at[pl.ds(cid * ACC_ROWS + tid * ZROWS, ZROWS)],
    )


def _deg_call(dstr, zeros1):
    return pl.kernel(
        _deg_body,
        out_type=jax.ShapeDtypeStruct((NC * ACC_ROWS,), jnp.float32),
        mesh=_mesh(),
        scratch_types=[
            pltpu.VMEM((LW,), jnp.float32),
            pltpu.VMEM((KC, LW), jnp.int32),
            pltpu.VMEM_SHARED((ACC_ROWS,), jnp.float32),
            pltpu.SemaphoreType.DMA,
        ],
        compiler_params=_SC_PARAMS,
        name="sc_deg_hist",
    )(dstr, zeros1)


def _agg1_body(t0, t1, t2, t3, srcr_hbm, dstr_hbm, zeros_hbm,
               o0, o1, o2, o3,
               idx_v, rows_v, acc, sem_i, sem_g, sem_sa, sem_sb):
    cid = lax.axis_index("c")
    tid = lax.axis_index("s")
    NIT = R_TILE // 4
    sem_s = (sem_sa, sem_sb)
    tabs = (t0, t1, t2, t3)
    outs = (o0, o1, o2, o3)

    def stage(X, w):
        base = tid * R_TILE + 2 * w
        pltpu.async_copy(srcr_hbm.at[pl.ds(base, 2)],
                         idx_v.at[X, pl.ds(0, 2)], sem_i)
        pltpu.async_copy(dstr_hbm.at[pl.ds(base, 2)],
                         idx_v.at[X, pl.ds(2, 2)], sem_i)

    def stage_wait(X):
        for k in range(2):
            pltpu.make_async_copy(
                srcr_hbm.at[pl.ds(0, 2)],
                idx_v.at[X, pl.ds(2 * k, 2)], sem_i).wait()

    def gathers(X, table):
        for k in range(2):
            pltpu.async_copy(
                table.at[idx_v.at[X, k]], rows_v.at[X, k], sem_g)

    def gathers_wait(X, table):
        for k in range(2):
            pltpu.make_async_copy(
                table.at[pl.ds(0, LW)], rows_v.at[X, k], sem_g).wait()

    def scatters(X):
        for k in range(2):
            pltpu.async_copy(
                rows_v.at[X, k], acc.at[idx_v.at[X, 2 + k]], sem_s[X],
                add=True)

    def scatters_wait(X, table):
        for k in range(2):
            pltpu.make_async_copy(
                table.at[pl.ds(0, LW)], rows_v.at[X, k], sem_s[X]).wait()

    def run_pass(table, out_hbm):
        stage(0, 0)
        stage_wait(0)
        gathers(0, table)
        stage(1, 1)

        def pipe(i, _):
            gathers_wait(0, table)
            scatters(0)

            @pl.when(i > 0)
            def _():
                scatters_wait(1, table)
                stage(1, 2 * i + 1)
            stage_wait(1)
            gathers(1, table)
            gathers_wait(1, table)
            scatters(1)
            scatters_wait(0, table)

            @pl.when(i < NIT - 1)
            def _():
                stage(0, 2 * i + 2)
                stage_wait(0)
                gathers(0, table)
            return 0
        lax.fori_loop(0, NIT, pipe, 0, unroll=False)
        scatters_wait(1, table)
        plsc.subcore_barrier()

        pltpu.sync_copy(
            acc.at[pl.ds(tid * OUT_TILE, OUT_TILE)],
            out_hbm.at[pl.ds(tid * OUT_TILE, OUT_TILE)],
        )

    for p in range(2):
        pltpu.sync_copy(zeros_hbm, acc.at[pl.ds(tid * ZROWS, ZROWS)])
        plsc.subcore_barrier()

        @pl.when(cid == 0)
        def _():
            run_pass(tabs[p], outs[p])

        @pl.when(cid == 1)
        def _():
            run_pass(tabs[2 + p], outs[2 + p])
        plsc.subcore_barrier()


def _agg1_call(h1s_tabs, srcr, dstr, zeros32):
    return pl.kernel(
        _agg1_body,
        out_type=[jax.ShapeDtypeStruct((N, 32), jnp.float32)] * 4,
        mesh=_mesh(),
        scratch_types=[
            pltpu.VMEM((2, 4, LW), jnp.int32),
            pltpu.VMEM((2, 2, LW, 32), jnp.float32),
            pltpu.VMEM_SHARED((ACC_ROWS, 32), jnp.float32),
            pltpu.SemaphoreType.DMA,
            pltpu.SemaphoreType.DMA,
            pltpu.SemaphoreType.DMA,
            pltpu.SemaphoreType.DMA,
        ],
        compiler_params=_SC_PARAMS,
        name="sc_agg1",
    )(*h1s_tabs, srcr, dstr, zeros32)


def _agg2_body(h2p_hbm, srcr_hbm, dstr_hbm, zeros_hbm, out_hbm,
               src_v, dst_v, rows_v, acc, sem, sem2):
    cid = lax.axis_index("c")
    tid = lax.axis_index("s")

    pltpu.sync_copy(zeros_hbm, acc.at[pl.ds(tid * ZROWS, ZROWS)])
    plsc.subcore_barrier()

    def chunk(ci, _):
        row0 = cid * (R // NC) + tid * R_HALF_TILE + ci * KC
        pltpu.sync_copy(srcr_hbm.at[pl.ds(row0, KC)], src_v)
        pltpu.sync_copy(dstr_hbm.at[pl.ds(row0, KC)], dst_v)
        cps = [
            pltpu.async_copy(h2p_hbm.at[src_v.at[j]], rows_v.at[j], sem)
            for j in range(KC)
        ]
        for cp in cps:
            cp.wait()
        sps = [
            pltpu.async_copy(rows_v.at[j], acc.at[dst_v.at[j]], sem2, add=True)
            for j in range(KC)
        ]
        for sp in sps:
            sp.wait()
        return 0
    lax.fori_loop(0, R_HALF_TILE // KC, chunk, 0, unroll=False)
    plsc.subcore_barrier()

    pltpu.sync_copy(
        acc.at[pl.ds(tid * OUT_TILE, OUT_TILE)],
        out_hbm.at[pl.ds(cid * N + tid * OUT_TILE, OUT_TILE)],
    )


def _agg2_call(h2p, srcr, dstr, zeros8):
    return pl.kernel(
        _agg2_body,
        out_type=jax.ShapeDtypeStruct((NC * N, 8), jnp.float32),
        mesh=_mesh(),
        scratch_types=[
            pltpu.VMEM((KC, LW), jnp.int32),
            pltpu.VMEM((KC, LW), jnp.int32),
            pltpu.VMEM((KC, LW, 8), jnp.float32),
            pltpu.VMEM_SHARED((ACC_ROWS, 8), jnp.float32),
            pltpu.SemaphoreType.DMA,
            pltpu.SemaphoreType.DMA,
        ],
        compiler_params=_SC_PARAMS,
        name="sc_agg2",
    )(h2p, srcr, dstr, zeros8)


def _mm1_body(x_ref, w_ref, o_ref):
    o_ref[...] = jnp.dot(x_ref[...], w_ref[...],
                         preferred_element_type=jnp.float32)


def _mm1_call(x, W1):
    return pl.pallas_call(
        _mm1_body,
        grid=(NB,),
        in_specs=[
            pl.BlockSpec((BM, F_IN), lambda i: (i, 0)),
            pl.BlockSpec((F_IN, H), lambda i: (0, 0)),
        ],
        out_specs=pl.BlockSpec((BM, H), lambda i: (i, 0)),
        out_shape=jax.ShapeDtypeStruct((N, H), jnp.float32),
        name="tc_mm1",
    )(x, W1)


def _scale_body(h1_ref, deg_ref, s0, s1, s2, s3, dinv_ref):
    deg = deg_ref[:, 0:1] + deg_ref[:, 1:2] + 1.0
    dv = lax.rsqrt(deg)
    dinv_ref[...] = dv
    hp = h1_ref[...] * dv
    for s, ref in enumerate((s0, s1, s2, s3)):
        ref[...] = hp[:, 32 * s:32 * s + 32]


def _scale_call(h1, deg2t):
    return pl.pallas_call(
        _scale_body,
        grid=(NB,),
        in_specs=[
            pl.BlockSpec((BM, H), lambda i: (i, 0)),
            pl.BlockSpec((BM, 2), lambda i: (i, 0)),
        ],
        out_specs=[pl.BlockSpec((BM, 32), lambda i: (i, 0))] * 4
        + [pl.BlockSpec((BM, 1), lambda i: (i, 0))],
        out_shape=[jax.ShapeDtypeStruct((N, 32), jnp.float32)] * 4
        + [jax.ShapeDtypeStruct((N, 1), jnp.float32)],
        name="tc_scale_slice",
    )(h1, deg2t)


def _layer2_body(a0, a1, a2, a3, g0, g1, g2, g3, dinv_ref, b1_ref, w2_ref,
                 o_ref):
    dv = dinv_ref[...]
    a = jnp.concatenate(
        [a0[...] + g0[...], a1[...] + g1[...],
         a2[...] + g2[...], a3[...] + g3[...]], axis=1)
    z1 = jnp.maximum(a * dv + b1_ref[...][None, :], 0.0)
    h2 = jnp.dot(z1, w2_ref[...], preferred_element_type=jnp.float32)
    o_ref[...] = h2 * dv


def _layer2_call(aggs, h1s_tabs, dinv, b1, W2p):
    return pl.pallas_call(
        _layer2_body,
        grid=(NB,),
        in_specs=(
            [pl.BlockSpec((BM, 32), lambda i: (i, 0))] * 8
            + [
                pl.BlockSpec((BM, 1), lambda i: (i, 0)),
                pl.BlockSpec((H,), lambda i: (0,)),
                pl.BlockSpec((H, 8), lambda i: (0, 0)),
            ]
        ),
        out_specs=pl.BlockSpec((BM, 8), lambda i: (i, 0)),
        out_shape=jax.ShapeDtypeStruct((N, 8), jnp.float32),
        name="tc_layer2",
    )(*aggs, *h1s_tabs, dinv, b1, W2p)


def _final_body(p0, p1, h2p_ref, dinv_ref, b2_ref, o_ref):
    s = p0[...] + p1[...] + h2p_ref[...]
    o_ref[...] = s * dinv_ref[...] + b2_ref[...][None, :]


def _final_call(agg2, h2p, dinv, b2p):
    def part_spec(c):
        return pl.BlockSpec((BM, 8), lambda i, c=c: (c * NB + i, 0))
    return pl.pallas_call(
        _final_body,
        grid=(NB,),
        in_specs=[
            part_spec(0),
            part_spec(1),
            pl.BlockSpec((BM, 8), lambda i: (i, 0)),
            pl.BlockSpec((BM, 1), lambda i: (i, 0)),
            pl.BlockSpec((8,), lambda i: (0,)),
        ],
        out_specs=pl.BlockSpec((BM, 8), lambda i: (i, 0)),
        out_shape=jax.ShapeDtypeStruct((N, 8), jnp.float32),
        name="tc_final",
    )(agg2, agg2, h2p, dinv, b2p)


def kernel(x, edge_index, y, W1, b1, W2, b2):
    ei = edge_index.astype(jnp.int32)
    pad = E_PAD - E
    srcr = jnp.concatenate(
        [ei[0], jnp.zeros((pad,), jnp.int32)]).reshape(R, LW)
    dstr = jnp.concatenate(
        [ei[1], jnp.full((pad,), N, jnp.int32)]).reshape(R, LW)
    W2p = jnp.pad(W2, ((0, 0), (0, 1)))
    b2p = jnp.pad(b2, (0, 1))
    zeros1 = jnp.zeros((ZROWS,), jnp.float32)
    zeros32 = jnp.zeros((ZROWS, 32), jnp.float32)
    zeros8 = jnp.zeros((ZROWS, 8), jnp.float32)

    h1 = _mm1_call(x, W1)
    degflat = _deg_call(dstr, zeros1)
    deg2t = degflat.reshape(NC, ACC_ROWS)[:, :N].T
    *h1s_tabs, dinv = _scale_call(h1, deg2t)
    aggs = _agg1_call(h1s_tabs, srcr, dstr, zeros32)
    h2p = _layer2_call(aggs, h1s_tabs, dinv, b1, W2p)
    agg2 = _agg2_call(h2p, srcr, dstr, zeros8)
    out8 = _final_call(agg2, h2p, dinv, b2p)
    return out8[:, :C]
